# baseline (speedup 1.0000x reference)
import functools

import jax
import jax.numpy as jnp
from jax.experimental import pallas as pl
from jax.experimental.pallas import tpu as pltpu

N = 10000
E = 160000
L = 320000
B = 256
D = 128
ED = 6
NITER = 4


def _pre_node_body(x_ref, wmlp_ref, bmlp_ref, wu_ref, wv_ref,
                   h_ref, eu_ref, ev_ref):
    h = jnp.dot(x_ref[...], wmlp_ref[...],
                preferred_element_type=jnp.float32) + bmlp_ref[...]
    h_ref[...] = h
    eu_ref[...] = jnp.dot(h, wu_ref[...], preferred_element_type=jnp.float32)
    ev_ref[...] = jnp.dot(h, wv_ref[...], preferred_element_type=jnp.float32)


def _pre_node(x, W_mlp, b_mlp, W_u, W_v):
    blk = 1000
    grid = (N // blk,)
    return pl.pallas_call(
        _pre_node_body,
        grid=grid,
        in_specs=[
            pl.BlockSpec((blk, D), lambda i: (i, 0)),
            pl.BlockSpec((D, D), lambda i: (0, 0)),
            pl.BlockSpec((1, D), lambda i: (0, 0)),
            pl.BlockSpec((D, D), lambda i: (0, 0)),
            pl.BlockSpec((D, D), lambda i: (0, 0)),
        ],
        out_specs=[
            pl.BlockSpec((blk, D), lambda i: (i, 0)),
            pl.BlockSpec((blk, D), lambda i: (i, 0)),
            pl.BlockSpec((blk, D), lambda i: (i, 0)),
        ],
        out_shape=[jax.ShapeDtypeStruct((N, D), jnp.float32)] * 3,
    )(x, W_mlp, b_mlp.reshape(1, D), W_u, W_v)


def _pre_edge_body(ea_ref, we_ref, euv_ref):
    euv_ref[...] = jnp.dot(ea_ref[...], we_ref[...],
                           preferred_element_type=jnp.float32)


def _pre_edge(edge_attr, W_edge):
    blk = 2000
    ea = jnp.pad(edge_attr, ((0, 0), (0, 8 - ED)))
    we = jnp.pad(W_edge, ((0, 8 - ED), (0, 0)))
    return pl.pallas_call(
        _pre_edge_body,
        grid=(E // blk,),
        in_specs=[
            pl.BlockSpec((blk, 8), lambda i: (i, 0)),
            pl.BlockSpec((8, D), lambda i: (0, 0)),
        ],
        out_specs=pl.BlockSpec((blk, D), lambda i: (i, 0)),
        out_shape=jax.ShapeDtypeStruct((E, D), jnp.float32),
    )(ea, we)


def _final_body(xo_ref, w_ref, b_ref, o_ref):
    o_ref[...] = jnp.dot(xo_ref[...], w_ref[...],
                         preferred_element_type=jnp.float32) + b_ref[...]


def _final_linear(xo, W_lb, b_lb):
    blk = 1000
    return pl.pallas_call(
        _final_body,
        grid=(N // blk,),
        in_specs=[
            pl.BlockSpec((blk, D), lambda i: (i, 0)),
            pl.BlockSpec((D, D), lambda i: (0, 0)),
            pl.BlockSpec((1, D), lambda i: (0, 0)),
        ],
        out_specs=pl.BlockSpec((blk, D), lambda i: (i, 0)),
        out_shape=jax.ShapeDtypeStruct((N, D), jnp.float32),
    )(xo, W_lb, b_lb.reshape(1, D))


def kernel(x, edge_attr, edge_index, line_graph_edge_index, edge_index_batch,
           W_mlp, b_mlp, W_u, W_v, W_edge, W_att_root, W_att_rel, b_att, a,
           W_gout, b_gout, a_bias, W_lb, b_lb):
    h, eu, ev = _pre_node(x, W_mlp, b_mlp, W_u, W_v)
    euv = _pre_edge(edge_attr, W_edge)

    src = edge_index[0]
    dst = edge_index[1]
    lg_src = line_graph_edge_index[0]
    lg_dst = line_graph_edge_index[1]
    batch = edge_index_batch

    e0 = (eu[src] + ev[dst] + euv) / 3.0
    out = e0
    out_list = []
    gout_list = []
    for n in range(NITER):
        agg = jax.ops.segment_sum(out[lg_src], lg_dst, num_segments=E)
        out = e0 + agg
        s = out @ W_att_rel
        nbs = jax.ops.segment_sum(s[lg_src], lg_dst, num_segments=E)
        xc = out @ W_att_root + nbs + b_att
        m = jax.ops.segment_max(xc, batch, num_segments=B)
        ex = jnp.exp(xc - m[batch])
        den = jax.ops.segment_sum(ex, batch, num_segments=B)
        sc = ex / den[batch]
        gx = jax.ops.segment_sum(out * sc, batch, num_segments=B)
        out_list.append(out)
        gout_list.append(jnp.tanh(gx @ W_gout + b_gout))
    gout_all = jnp.stack(gout_list, axis=-1)
    out_all = jnp.stack(out_list, axis=-1)
    scores = jnp.sum(gout_all * a, axis=1, keepdims=True) + a_bias
    scores = jax.nn.softmax(scores, axis=-1)
    scores_e = scores[batch]
    out_fin = jnp.sum(out_all * scores_e, axis=-1)
    xo = h + jax.ops.segment_sum(out_fin, dst, num_segments=N)
    return _final_linear(xo, W_lb, b_lb)
